# baseline (speedup 1.0000x reference)
import functools

import jax
import jax.numpy as jnp
from jax import lax
from jax.experimental import pallas as pl
from jax.experimental.pallas import tpu as pltpu
from jax.experimental.pallas import tpu_sc as plsc

FG_IOU = 0.7
BG_IOU = 0.3
LN2 = 0.6931471805599453


def _tc_body(aT_ref, gt_ref, num_ref, img_ref, am_ref, aarg_ref,
             *, NP, L, GS):
    b = pl.program_id(0)
    img_h = img_ref[0, 0]
    img_w = img_ref[0, 1]
    n_gt = num_ref[b]

    g = gt_ref[0]
    gx1 = g[:, 0:1]
    gy1 = g[:, 1:2]
    gx2 = g[:, 2:3]
    gy2 = g[:, 3:4]
    gw = gx2 - gx1 + 1.0
    gh = gy2 - gy1 + 1.0
    garea = gw * gh
    gidx = jax.lax.broadcasted_iota(jnp.int32, (GS, 1), 0)
    gvalid = gidx < n_gt

    lane_i = jax.lax.broadcasted_iota(jnp.int32, (GS, L), 1)
    g_i = jax.lax.broadcasted_iota(jnp.int32, (GS, L), 0)

    nch = NP // L
    acc_cmax = jnp.full((GS, 1), -3.0, jnp.float32)
    acc_carg = jnp.zeros((GS, 1), jnp.int32)
    row_max = []
    row_arg = []

    for c in range(nch):
        off = c * L
        ax1 = aT_ref[0:1, off:off + L]
        ay1 = aT_ref[1:2, off:off + L]
        ax2 = aT_ref[2:3, off:off + L]
        ay2 = aT_ref[3:4, off:off + L]
        aw = ax2 - ax1 + 1.0
        ah = ay2 - ay1 + 1.0
        keep = (ax1 >= 0.0) & (ay1 >= 0.0) & (ax2 < img_w) & (ay2 < img_h)
        aarea = aw * ah
        ix1 = jnp.maximum(ax1, gx1)
        iy1 = jnp.maximum(ay1, gy1)
        ix2 = jnp.minimum(ax2, gx2)
        iy2 = jnp.minimum(ay2, gy2)
        iw = jnp.clip(ix2 - ix1 + 1.0, 0.0)
        ih = jnp.clip(iy2 - iy1 + 1.0, 0.0)
        inter = iw * ih
        iou = inter / (aarea + garea - inter)
        ov = jnp.where(keep, iou, -1.0)
        cm = jnp.max(ov, axis=1, keepdims=True)
        carg = jnp.min(jnp.where(ov == cm, lane_i, NP), axis=1,
                       keepdims=True) + off
        better = cm > acc_cmax
        acc_carg = jnp.where(better, carg, acc_carg)
        acc_cmax = jnp.maximum(acc_cmax, cm)
        am = jnp.max(ov, axis=0, keepdims=True)
        aarg = jnp.min(jnp.where(ov == am, g_i, GS), axis=0, keepdims=True)
        row_max.append(am)
        row_arg.append(aarg)

    acc_carg_m = jnp.where(gvalid, acc_carg, -1)
    for c in range(nch):
        off = c * L
        eq = (acc_carg_m - off) == lane_i
        best_g = jnp.max(jnp.where(eq, g_i, -1), axis=0, keepdims=True)
        override = best_g >= 0
        am_ref[0, 0:1, off:off + L] = jnp.where(override, 2.0, row_max[c])
        aarg_ref[0, 0:1, off:off + L] = jnp.where(override, best_g,
                                                  row_arg[c])


def _plog(x):
    bits = jax.lax.bitcast_convert_type(x, jnp.int32)
    e = ((bits >> 23) & 0xFF) - 127
    m = jax.lax.bitcast_convert_type(
        (bits & 0x7FFFFF) | 0x3F800000, jnp.float32)
    big = m > 1.4142135
    m = jnp.where(big, m * 0.5, m)
    e = e + jnp.where(big, 1, 0)
    z = (m - 1.0) / (m + 1.0)
    z2 = z * z
    p = z * (2.0 + z2 * (2.0 / 3.0 + z2 * (2.0 / 5.0 + z2 * (
        2.0 / 7.0 + z2 * (2.0 / 9.0 + z2 * (2.0 / 11.0))))))
    return p + e.astype(jnp.float32) * LN2


def _sc_body(aT, am_h, aarg_h, gx1_h, gy1_h, gx2_h, gy2_h, glab_h,
             imw_h, imh_h, cls_h, reg_h,
             ax1v, ay1v, ax2v, ay2v, amv, agv,
             tx1, ty1, tx2, ty2, tlab, clsv, t0v, t1v, t2v, t3v,
             imwv, imhv, *, B, NP, SL, NC):
    wid = lax.axis_index("s") * NC + lax.axis_index("c")
    base = wid * SL
    pltpu.sync_copy(aT.at[0, pl.ds(base, SL)], ax1v)
    pltpu.sync_copy(aT.at[1, pl.ds(base, SL)], ay1v)
    pltpu.sync_copy(aT.at[2, pl.ds(base, SL)], ax2v)
    pltpu.sync_copy(aT.at[3, pl.ds(base, SL)], ay2v)
    pltpu.sync_copy(imw_h, imwv)
    wvec = imwv[...]
    pltpu.sync_copy(imh_h, imhv)
    hvec = imhv[...]

    for b in range(B):
        pltpu.sync_copy(am_h.at[b, pl.ds(base, SL)], amv)
        pltpu.sync_copy(aarg_h.at[b, pl.ds(base, SL)], agv)
        pltpu.sync_copy(gx1_h.at[b], tx1)
        pltpu.sync_copy(gy1_h.at[b], ty1)
        pltpu.sync_copy(gx2_h.at[b], tx2)
        pltpu.sync_copy(gy2_h.at[b], ty2)
        pltpu.sync_copy(glab_h.at[b], tlab)

        def do_vreg(i):
            sl = pl.ds(i * 16, 16)
            x1 = ax1v[sl]
            y1 = ay1v[sl]
            x2 = ax2v[sl]
            y2 = ay2v[sl]
            am16 = amv[sl]
            ag16 = agv[sl]
            gx1g = plsc.load_gather(tx1, [ag16])
            gy1g = plsc.load_gather(ty1, [ag16])
            gx2g = plsc.load_gather(tx2, [ag16])
            gy2g = plsc.load_gather(ty2, [ag16])
            labg = plsc.load_gather(tlab, [ag16])
            aw = x2 - x1 + 1.0
            ah = y2 - y1 + 1.0
            keep = (x1 >= 0.0) & (y1 >= 0.0) & (x2 < wvec) & (y2 < hvec)
            gwv = gx2g - gx1g + 1.0
            ghv = gy2g - gy1g + 1.0
            gcx = gx1g + 0.5 * gwv
            gcy = gy1g + 0.5 * ghv
            acx = x1 + 0.5 * aw
            acy = y1 + 0.5 * ah
            tx = ((gcx - acx) / aw) / 0.1
            ty = ((gcy - acy) / ah) / 0.1
            tw = _plog(gwv / aw) / 0.2
            th = _plog(ghv / ah) / 0.2
            cls = jnp.where(am16 < FG_IOU, 0.0, labg)
            cls = jnp.where((am16 < FG_IOU) & (am16 > BG_IOU), -1.0, cls)
            clsv[sl] = jnp.where(keep, cls, -1.0)
            t0v[sl] = jnp.where(keep, tx, 0.0)
            t1v[sl] = jnp.where(keep, ty, 0.0)
            t2v[sl] = jnp.where(keep, tw, 0.0)
            t3v[sl] = jnp.where(keep, th, 0.0)

        def inner(i0, _):
            for u in range(4):
                do_vreg(i0 * 4 + u)
            return 0

        lax.fori_loop(0, SL // 64, inner, 0)
        pltpu.sync_copy(clsv, cls_h.at[b, pl.ds(base, SL)])
        pltpu.sync_copy(t0v, reg_h.at[b, 0, pl.ds(base, SL)])
        pltpu.sync_copy(t1v, reg_h.at[b, 1, pl.ds(base, SL)])
        pltpu.sync_copy(t2v, reg_h.at[b, 2, pl.ds(base, SL)])
        pltpu.sync_copy(t3v, reg_h.at[b, 3, pl.ds(base, SL)])


def kernel(anchors, gt_boxes, img_info, num_gt_boxes):
    N = anchors.shape[0]
    B, G = gt_boxes.shape[0], gt_boxes.shape[1]
    L = 2048
    NP = ((N + L - 1) // L) * L
    GS = ((G + 7) // 8) * 8
    if GS == G:
        GS = G + 8

    pad = jnp.tile(jnp.array([[0.0, 0.0, 2e9, 2e9]], jnp.float32),
                   (NP - N, 1))
    aT = jnp.concatenate([anchors.astype(jnp.float32), pad], axis=0).T
    aT = jnp.concatenate([aT, jnp.zeros((4, NP), jnp.float32)], axis=0)

    num = num_gt_boxes.astype(jnp.int32)
    img = img_info.astype(jnp.float32)

    gtp = jnp.pad(gt_boxes.astype(jnp.float32),
                  ((0, 0), (0, GS - G), (0, 8 - gt_boxes.shape[2])))
    valid = (jnp.arange(GS)[None, :] < num[:, None])[..., None]
    gts = jnp.where(valid, gtp, jnp.float32(-1e8))

    am, aarg = pl.pallas_call(
        functools.partial(_tc_body, NP=NP, L=L, GS=GS),
        grid=(B,),
        in_specs=[
            pl.BlockSpec((8, NP), lambda b: (0, 0)),
            pl.BlockSpec((1, GS, 8), lambda b: (b, 0, 0)),
            pl.BlockSpec(memory_space=pltpu.SMEM),
            pl.BlockSpec(memory_space=pltpu.SMEM),
        ],
        out_specs=[
            pl.BlockSpec((1, 1, NP), lambda b: (b, 0, 0)),
            pl.BlockSpec((1, 1, NP), lambda b: (b, 0, 0)),
        ],
        out_shape=[
            jax.ShapeDtypeStruct((B, 1, NP), jnp.float32),
            jax.ShapeDtypeStruct((B, 1, NP), jnp.int32),
        ],
        compiler_params=pltpu.CompilerParams(
            dimension_semantics=("parallel",)),
    )(aT, gts, num, img)

    am2 = am.reshape(B, NP)
    aarg2 = aarg.reshape(B, NP)
    gpad = jnp.pad(gtp, ((0, 0), (0, 64 - GS), (0, 0)))
    gx1a = gpad[:, :, 0]
    gy1a = gpad[:, :, 1]
    gx2a = gpad[:, :, 2]
    gy2a = gpad[:, :, 3]
    glaba = gpad[:, :, 4]
    imw16 = jnp.full((16,), img[0, 1], jnp.float32)
    imh16 = jnp.full((16,), img[0, 0], jnp.float32)

    NC, NS = 2, 16
    NW = NC * NS
    SL = NP // NW
    mesh = plsc.VectorSubcoreMesh(core_axis_name="c", subcore_axis_name="s",
                                  num_cores=NC)

    sck = pl.kernel(
        functools.partial(_sc_body, B=B, NP=NP, SL=SL, NC=NC),
        mesh=mesh,
        out_type=[
            jax.ShapeDtypeStruct((B, NP), jnp.float32),
            jax.ShapeDtypeStruct((B, 4, NP), jnp.float32),
        ],
        scratch_types=[
            pltpu.VMEM((SL,), jnp.float32),
            pltpu.VMEM((SL,), jnp.float32),
            pltpu.VMEM((SL,), jnp.float32),
            pltpu.VMEM((SL,), jnp.float32),
            pltpu.VMEM((SL,), jnp.float32),
            pltpu.VMEM((SL,), jnp.int32),
            pltpu.VMEM((64,), jnp.float32),
            pltpu.VMEM((64,), jnp.float32),
            pltpu.VMEM((64,), jnp.float32),
            pltpu.VMEM((64,), jnp.float32),
            pltpu.VMEM((64,), jnp.float32),
            pltpu.VMEM((SL,), jnp.float32),
            pltpu.VMEM((SL,), jnp.float32),
            pltpu.VMEM((SL,), jnp.float32),
            pltpu.VMEM((SL,), jnp.float32),
            pltpu.VMEM((SL,), jnp.float32),
            pltpu.VMEM((16,), jnp.float32),
            pltpu.VMEM((16,), jnp.float32),
        ],
        compiler_params=pltpu.CompilerParams(needs_layout_passes=False),
    )
    cls_f, reg4 = sck(aT, am2, aarg2, gx1a, gy1a, gx2a, gy2a, glaba,
                      imw16, imh16)

    cls = cls_f[:, :N]
    reg = jnp.transpose(reg4[:, :, :N], (0, 2, 1))
    return (cls, reg)


# speedup vs baseline: 1.7666x; 1.7666x over previous
import functools

import jax
import jax.numpy as jnp
from jax.experimental import pallas as pl
from jax.experimental.pallas import tpu as pltpu

FG_IOU = 0.7
BG_IOU = 0.3


def _body(aT_ref, gt_ref, gtd_ref, num_ref, img_ref, out_ref, *, NP, L, GS):
    b = pl.program_id(0)
    img_h = img_ref[0, 0]
    img_w = img_ref[0, 1]
    n_gt = num_ref[b]

    g = gt_ref[0]
    gx1 = g[:, 0:1]
    gy1 = g[:, 1:2]
    gx2 = g[:, 2:3]
    gy2 = g[:, 3:4]
    gw = gx2 - gx1 + 1.0
    gh = gy2 - gy1 + 1.0
    garea = gw * gh
    gd = gtd_ref[0]
    gidx = jax.lax.broadcasted_iota(jnp.int32, (GS, 1), 0)
    gvalid = gidx < n_gt

    lane_i = jax.lax.broadcasted_iota(jnp.int32, (GS, L), 1)
    g_i = jax.lax.broadcasted_iota(jnp.int32, (GS, L), 0)

    nch = NP // L
    acc_cmax = jnp.full((GS, 1), -3.0, jnp.float32)
    acc_carg = jnp.zeros((GS, 1), jnp.int32)
    row_max = []
    row_arg = []

    def anchor_chunk(off):
        ax1 = aT_ref[0:1, off:off + L]
        ay1 = aT_ref[1:2, off:off + L]
        ax2 = aT_ref[2:3, off:off + L]
        ay2 = aT_ref[3:4, off:off + L]
        aw = ax2 - ax1 + 1.0
        ah = ay2 - ay1 + 1.0
        keep = (ax1 >= 0.0) & (ay1 >= 0.0) & (ax2 < img_w) & (ay2 < img_h)
        return ax1, ay1, ax2, ay2, aw, ah, keep

    for c in range(nch):
        off = c * L
        ax1, ay1, ax2, ay2, aw, ah, keep = anchor_chunk(off)
        aarea = aw * ah
        ix1 = jnp.maximum(ax1, gx1)
        iy1 = jnp.maximum(ay1, gy1)
        ix2 = jnp.minimum(ax2, gx2)
        iy2 = jnp.minimum(ay2, gy2)
        iw = jnp.clip(ix2 - ix1 + 1.0, 0.0)
        ih = jnp.clip(iy2 - iy1 + 1.0, 0.0)
        inter = iw * ih
        iou = inter / (aarea + garea - inter)
        ov = jnp.where(keep, iou, -1.0)
        cm = jnp.max(ov, axis=1, keepdims=True)
        carg = jnp.min(jnp.where(ov == cm, lane_i, NP), axis=1,
                       keepdims=True) + off
        better = cm > acc_cmax
        acc_carg = jnp.where(better, carg, acc_carg)
        acc_cmax = jnp.maximum(acc_cmax, cm)
        am = jnp.max(ov, axis=0, keepdims=True)
        aarg = jnp.min(jnp.where(ov == am, g_i, GS), axis=0, keepdims=True)
        row_max.append(am)
        row_arg.append(aarg)

    acc_carg_m = jnp.where(gvalid, acc_carg, -1)

    for c in range(nch):
        off = c * L
        am = row_max[c]
        aarg = row_arg[c]
        eq = (acc_carg_m - off) == lane_i
        best_g = jnp.max(jnp.where(eq, g_i, -1), axis=0, keepdims=True)
        override = best_g >= 0
        arg_f = jnp.where(override, best_g, aarg)
        max_f = jnp.where(override, 2.0, am)
        onehot = (g_i == arg_f).astype(jnp.bfloat16)
        gat = jax.lax.dot_general(gd, onehot, (((1,), (0,)), ((), ())),
                                  preferred_element_type=jnp.float32)
        s_gx1 = gat[0:1, :] + gat[1:2, :] + gat[2:3, :]
        s_gy1 = gat[3:4, :] + gat[4:5, :] + gat[5:6, :]
        s_gx2 = gat[6:7, :] + gat[7:8, :] + gat[8:9, :]
        s_gy2 = gat[9:10, :] + gat[10:11, :] + gat[11:12, :]
        label = gat[12:13, :]
        s_gw = s_gx2 - s_gx1 + 1.0
        s_gh = s_gy2 - s_gy1 + 1.0
        s_gcx = s_gx1 + 0.5 * s_gw
        s_gcy = s_gy1 + 0.5 * s_gh
        ax1, ay1, ax2, ay2, aw, ah, keep = anchor_chunk(off)
        acx = ax1 + 0.5 * aw
        acy = ay1 + 0.5 * ah
        tx = ((s_gcx - acx) / aw) / 0.1
        ty = ((s_gcy - acy) / ah) / 0.1
        tw = jnp.log(s_gw / aw) / 0.2
        th = jnp.log(s_gh / ah) / 0.2
        cls = jnp.where(max_f < FG_IOU, 0.0, label)
        cls = jnp.where((max_f < FG_IOU) & (max_f > BG_IOU), -1.0, cls)
        cls = jnp.where(keep, cls, -1.0)
        out_ref[0, 0:1, off:off + L] = cls
        out_ref[0, 1:2, off:off + L] = jnp.where(keep, tx, 0.0)
        out_ref[0, 2:3, off:off + L] = jnp.where(keep, ty, 0.0)
        out_ref[0, 3:4, off:off + L] = jnp.where(keep, tw, 0.0)
        out_ref[0, 4:5, off:off + L] = jnp.where(keep, th, 0.0)


def _bf16_split3(x):
    def trunc(v):
        bits = jax.lax.bitcast_convert_type(v, jnp.uint32)
        return jax.lax.bitcast_convert_type(
            bits & jnp.uint32(0xFFFF0000), jnp.float32)
    hi = trunc(x)
    r = x - hi
    mid = trunc(r)
    lo = r - mid
    return hi, mid, lo


def kernel(anchors, gt_boxes, img_info, num_gt_boxes):
    N = anchors.shape[0]
    B, G = gt_boxes.shape[0], gt_boxes.shape[1]
    L = 4096
    NP = ((N + L - 1) // L) * L
    GS = ((G + 7) // 8) * 8
    if GS == G:
        GS = G + 8

    pad = jnp.tile(jnp.array([[0.0, 0.0, 2e9, 2e9]], jnp.float32),
                   (NP - N, 1))
    aT = jnp.concatenate([anchors.astype(jnp.float32), pad], axis=0).T
    aT = jnp.concatenate([aT, jnp.zeros((4, NP), jnp.float32)], axis=0)

    num = num_gt_boxes.astype(jnp.int32)
    img = img_info.astype(jnp.float32)

    gtp = jnp.pad(gt_boxes.astype(jnp.float32),
                  ((0, 0), (0, GS - G), (0, 8 - gt_boxes.shape[2])))
    valid = (jnp.arange(GS)[None, :] < num[:, None])[..., None]
    gts = jnp.where(valid, gtp, jnp.float32(-1e8))

    hi, mid, lo = _bf16_split3(gtp[:, :, :4])
    rows = [hi[:, :, 0], mid[:, :, 0], lo[:, :, 0],
            hi[:, :, 1], mid[:, :, 1], lo[:, :, 1],
            hi[:, :, 2], mid[:, :, 2], lo[:, :, 2],
            hi[:, :, 3], mid[:, :, 3], lo[:, :, 3],
            gtp[:, :, 4]]
    gtd = jnp.stack(rows, axis=1)
    gtd = jnp.pad(gtd, ((0, 0), (0, 16 - gtd.shape[1]), (0, 0)))
    gtd = gtd.astype(jnp.bfloat16)

    out = pl.pallas_call(
        functools.partial(_body, NP=NP, L=L, GS=GS),
        grid=(B,),
        in_specs=[
            pl.BlockSpec((8, NP), lambda b: (0, 0)),
            pl.BlockSpec((1, GS, 8), lambda b: (b, 0, 0)),
            pl.BlockSpec((1, 16, GS), lambda b: (b, 0, 0)),
            pl.BlockSpec(memory_space=pltpu.SMEM),
            pl.BlockSpec(memory_space=pltpu.SMEM),
        ],
        out_specs=pl.BlockSpec((1, 8, NP), lambda b: (b, 0, 0)),
        out_shape=jax.ShapeDtypeStruct((B, 8, NP), jnp.float32),
        compiler_params=pltpu.CompilerParams(
            dimension_semantics=("parallel",)),
    )(aT, gts, gtd, num, img)

    cls = out[:, 0, :N]
    reg = jnp.transpose(out[:, 1:5, :N], (0, 2, 1))
    return (cls, reg)


# speedup vs baseline: 1.7850x; 1.0104x over previous
import functools

import jax
import jax.numpy as jnp
from jax.experimental import pallas as pl
from jax.experimental.pallas import tpu as pltpu

FG_IOU = 0.7
BG_IOU = 0.3


def _body(aT_ref, gt_ref, gtd_ref, num_ref, img_ref, out_ref, *, NP, L, GS):
    b = pl.program_id(0)
    img_h = img_ref[0, 0]
    img_w = img_ref[0, 1]
    n_gt = num_ref[b]

    g = gt_ref[0]
    gx1 = g[:, 0:1]
    gy1 = g[:, 1:2]
    gx2 = g[:, 2:3]
    gy2 = g[:, 3:4]
    gw = gx2 - gx1 + 1.0
    gh = gy2 - gy1 + 1.0
    garea = gw * gh
    gd = gtd_ref[0]
    gidx = jax.lax.broadcasted_iota(jnp.int32, (GS, 1), 0)
    gvalid = gidx < n_gt

    lane_i = jax.lax.broadcasted_iota(jnp.int32, (GS, L), 1)
    g_i = jax.lax.broadcasted_iota(jnp.int32, (GS, L), 0)

    nch = NP // L
    acc_cmax = jnp.full((GS, 1), -3.0, jnp.float32)
    acc_carg = jnp.zeros((GS, 1), jnp.int32)
    row_max = []
    row_arg = []

    def anchor_chunk(off):
        ax1 = aT_ref[0:1, off:off + L]
        ay1 = aT_ref[1:2, off:off + L]
        ax2 = aT_ref[2:3, off:off + L]
        ay2 = aT_ref[3:4, off:off + L]
        aw = ax2 - ax1 + 1.0
        ah = ay2 - ay1 + 1.0
        keep = (ax1 >= 0.0) & (ay1 >= 0.0) & (ax2 < img_w) & (ay2 < img_h)
        return ax1, ay1, ax2, ay2, aw, ah, keep

    for c in range(nch):
        off = c * L
        ax1, ay1, ax2, ay2, aw, ah, keep = anchor_chunk(off)
        aarea = aw * ah
        ix1 = jnp.maximum(ax1, gx1)
        iy1 = jnp.maximum(ay1, gy1)
        ix2 = jnp.minimum(ax2, gx2)
        iy2 = jnp.minimum(ay2, gy2)
        iw = jnp.clip(ix2 - ix1 + 1.0, 0.0)
        ih = jnp.clip(iy2 - iy1 + 1.0, 0.0)
        inter = iw * ih
        iou = inter / (aarea + garea - inter)
        ov = jnp.where(keep, iou, -1.0)
        cm = jnp.max(ov, axis=1, keepdims=True)
        carg = jnp.min(jnp.where(ov == cm, lane_i, NP), axis=1,
                       keepdims=True) + off
        better = cm > acc_cmax
        acc_carg = jnp.where(better, carg, acc_carg)
        acc_cmax = jnp.maximum(acc_cmax, cm)
        am = jnp.max(ov, axis=0, keepdims=True)
        aarg = jnp.min(jnp.where(ov == am, g_i, GS), axis=0, keepdims=True)
        row_max.append(am)
        row_arg.append(aarg)

    acc_carg_m = jnp.where(gvalid, acc_carg, -1)

    for c in range(nch):
        off = c * L
        am = row_max[c]
        aarg = row_arg[c]
        eq = (acc_carg_m - off) == lane_i
        best_g = jnp.max(jnp.where(eq, g_i, -1), axis=0, keepdims=True)
        override = best_g >= 0
        arg_f = jnp.where(override, best_g, aarg)
        max_f = jnp.where(override, 2.0, am)
        onehot = (g_i == arg_f).astype(jnp.bfloat16)
        gat = jax.lax.dot_general(gd, onehot, (((1,), (0,)), ((), ())),
                                  preferred_element_type=jnp.float32)
        s_gx1 = gat[0:1, :] + gat[1:2, :] + gat[2:3, :]
        s_gy1 = gat[3:4, :] + gat[4:5, :] + gat[5:6, :]
        s_gx2 = gat[6:7, :] + gat[7:8, :] + gat[8:9, :]
        s_gy2 = gat[9:10, :] + gat[10:11, :] + gat[11:12, :]
        label = gat[12:13, :]
        s_gw = s_gx2 - s_gx1 + 1.0
        s_gh = s_gy2 - s_gy1 + 1.0
        s_gcx = s_gx1 + 0.5 * s_gw
        s_gcy = s_gy1 + 0.5 * s_gh
        ax1, ay1, ax2, ay2, aw, ah, keep = anchor_chunk(off)
        acx = ax1 + 0.5 * aw
        acy = ay1 + 0.5 * ah
        tx = ((s_gcx - acx) / aw) / 0.1
        ty = ((s_gcy - acy) / ah) / 0.1
        tw = jnp.log(s_gw / aw) / 0.2
        th = jnp.log(s_gh / ah) / 0.2
        cls = jnp.where(max_f < FG_IOU, 0.0, label)
        cls = jnp.where((max_f < FG_IOU) & (max_f > BG_IOU), -1.0, cls)
        cls = jnp.where(keep, cls, -1.0)
        out_ref[0, 0:1, off:off + L] = cls
        out_ref[0, 1:2, off:off + L] = jnp.where(keep, tx, 0.0)
        out_ref[0, 2:3, off:off + L] = jnp.where(keep, ty, 0.0)
        out_ref[0, 3:4, off:off + L] = jnp.where(keep, tw, 0.0)
        out_ref[0, 4:5, off:off + L] = jnp.where(keep, th, 0.0)


def _bf16_split3(x):
    def trunc(v):
        bits = jax.lax.bitcast_convert_type(v, jnp.uint32)
        return jax.lax.bitcast_convert_type(
            bits & jnp.uint32(0xFFFF0000), jnp.float32)
    hi = trunc(x)
    r = x - hi
    mid = trunc(r)
    lo = r - mid
    return hi, mid, lo


def kernel(anchors, gt_boxes, img_info, num_gt_boxes):
    N = anchors.shape[0]
    B, G = gt_boxes.shape[0], gt_boxes.shape[1]
    L = 1024
    NP = ((N + L - 1) // L) * L
    GS = ((G + 7) // 8) * 8
    if GS == G:
        GS = G + 8

    pad = jnp.tile(jnp.array([[0.0, 0.0, 2e9, 2e9]], jnp.float32),
                   (NP - N, 1))
    aT = jnp.concatenate([anchors.astype(jnp.float32), pad], axis=0).T
    aT = jnp.concatenate([aT, jnp.zeros((4, NP), jnp.float32)], axis=0)

    num = num_gt_boxes.astype(jnp.int32)
    img = img_info.astype(jnp.float32)

    gtp = jnp.pad(gt_boxes.astype(jnp.float32),
                  ((0, 0), (0, GS - G), (0, 8 - gt_boxes.shape[2])))
    valid = (jnp.arange(GS)[None, :] < num[:, None])[..., None]
    gts = jnp.where(valid, gtp, jnp.float32(-1e8))

    hi, mid, lo = _bf16_split3(gtp[:, :, :4])
    rows = [hi[:, :, 0], mid[:, :, 0], lo[:, :, 0],
            hi[:, :, 1], mid[:, :, 1], lo[:, :, 1],
            hi[:, :, 2], mid[:, :, 2], lo[:, :, 2],
            hi[:, :, 3], mid[:, :, 3], lo[:, :, 3],
            gtp[:, :, 4]]
    gtd = jnp.stack(rows, axis=1)
    gtd = jnp.pad(gtd, ((0, 0), (0, 16 - gtd.shape[1]), (0, 0)))
    gtd = gtd.astype(jnp.bfloat16)

    out = pl.pallas_call(
        functools.partial(_body, NP=NP, L=L, GS=GS),
        grid=(B,),
        in_specs=[
            pl.BlockSpec((8, NP), lambda b: (0, 0)),
            pl.BlockSpec((1, GS, 8), lambda b: (b, 0, 0)),
            pl.BlockSpec((1, 16, GS), lambda b: (b, 0, 0)),
            pl.BlockSpec(memory_space=pltpu.SMEM),
            pl.BlockSpec(memory_space=pltpu.SMEM),
        ],
        out_specs=pl.BlockSpec((1, 8, NP), lambda b: (b, 0, 0)),
        out_shape=jax.ShapeDtypeStruct((B, 8, NP), jnp.float32),
        compiler_params=pltpu.CompilerParams(
            dimension_semantics=("parallel",)),
    )(aT, gts, gtd, num, img)

    cls = out[:, 0, :N]
    reg = jnp.transpose(out[:, 1:5, :N], (0, 2, 1))
    return (cls, reg)


# speedup vs baseline: 1.7912x; 1.0035x over previous
import functools

import jax
import jax.numpy as jnp
from jax.experimental import pallas as pl
from jax.experimental.pallas import tpu as pltpu

FG_IOU = 0.7
BG_IOU = 0.3


def _body(aT_ref, gt_ref, gtd_ref, num_ref, img_ref, out_ref, *, NP, L, GS):
    b = pl.program_id(0)
    img_h = img_ref[0, 0]
    img_w = img_ref[0, 1]
    n_gt = num_ref[b]

    g = gt_ref[0]
    gx1 = g[:, 0:1]
    gy1 = g[:, 1:2]
    gx2 = g[:, 2:3]
    gy2 = g[:, 3:4]
    gw = gx2 - gx1 + 1.0
    gh = gy2 - gy1 + 1.0
    garea = gw * gh
    gd = gtd_ref[0]
    gidx = jax.lax.broadcasted_iota(jnp.int32, (GS, 1), 0)
    gvalid = gidx < n_gt

    lane_i = jax.lax.broadcasted_iota(jnp.int32, (GS, L), 1)
    g_i = jax.lax.broadcasted_iota(jnp.int32, (GS, L), 0)

    nch = NP // L
    acc_cmax = jnp.full((GS, 1), -3.0, jnp.float32)
    acc_carg = jnp.zeros((GS, 1), jnp.int32)
    row_max = []
    row_arg = []

    def anchor_chunk(off):
        ax1 = aT_ref[0:1, off:off + L]
        ay1 = aT_ref[1:2, off:off + L]
        ax2 = aT_ref[2:3, off:off + L]
        ay2 = aT_ref[3:4, off:off + L]
        aw = ax2 - ax1 + 1.0
        ah = ay2 - ay1 + 1.0
        keep = (ax1 >= 0.0) & (ay1 >= 0.0) & (ax2 < img_w) & (ay2 < img_h)
        return ax1, ay1, ax2, ay2, aw, ah, keep

    for c in range(nch):
        off = c * L
        ax1, ay1, ax2, ay2, aw, ah, keep = anchor_chunk(off)
        aarea = aw * ah
        ix1 = jnp.maximum(ax1, gx1)
        iy1 = jnp.maximum(ay1, gy1)
        ix2 = jnp.minimum(ax2, gx2)
        iy2 = jnp.minimum(ay2, gy2)
        iw = jnp.clip(ix2 - ix1 + 1.0, 0.0)
        ih = jnp.clip(iy2 - iy1 + 1.0, 0.0)
        inter = iw * ih
        iou = inter / (aarea + garea - inter)
        ov = jnp.where(keep, iou, -1.0)
        cm = jnp.max(ov, axis=1, keepdims=True)
        carg = jnp.min(jnp.where(ov == cm, lane_i, NP), axis=1,
                       keepdims=True) + off
        better = cm > acc_cmax
        acc_carg = jnp.where(better, carg, acc_carg)
        acc_cmax = jnp.maximum(acc_cmax, cm)
        am = jnp.max(ov, axis=0, keepdims=True)
        aarg = jnp.min(jnp.where(ov == am, g_i, GS), axis=0, keepdims=True)
        row_max.append(am)
        row_arg.append(aarg)

    acc_carg_m = jnp.where(gvalid, acc_carg, -1)

    for c in range(nch):
        off = c * L
        am = row_max[c]
        aarg = row_arg[c]
        eq = (acc_carg_m - off) == lane_i
        best_g = jnp.max(jnp.where(eq, g_i, -1), axis=0, keepdims=True)
        override = best_g >= 0
        arg_f = jnp.where(override, best_g, aarg)
        max_f = jnp.where(override, 2.0, am)
        onehot = (g_i == arg_f).astype(jnp.bfloat16)
        gat = jax.lax.dot_general(gd, onehot, (((1,), (0,)), ((), ())),
                                  preferred_element_type=jnp.float32)
        s_gx1 = gat[0:1, :] + gat[1:2, :] + gat[2:3, :]
        s_gy1 = gat[3:4, :] + gat[4:5, :] + gat[5:6, :]
        s_gx2 = gat[6:7, :] + gat[7:8, :] + gat[8:9, :]
        s_gy2 = gat[9:10, :] + gat[10:11, :] + gat[11:12, :]
        label = gat[12:13, :]
        s_gw = s_gx2 - s_gx1 + 1.0
        s_gh = s_gy2 - s_gy1 + 1.0
        s_gcx = s_gx1 + 0.5 * s_gw
        s_gcy = s_gy1 + 0.5 * s_gh
        ax1, ay1, ax2, ay2, aw, ah, keep = anchor_chunk(off)
        acx = ax1 + 0.5 * aw
        acy = ay1 + 0.5 * ah
        tx = ((s_gcx - acx) / aw) / 0.1
        ty = ((s_gcy - acy) / ah) / 0.1
        tw = jnp.log(s_gw / aw) / 0.2
        th = jnp.log(s_gh / ah) / 0.2
        cls = jnp.where(max_f < FG_IOU, 0.0, label)
        cls = jnp.where((max_f < FG_IOU) & (max_f > BG_IOU), -1.0, cls)
        cls = jnp.where(keep, cls, -1.0)
        out_ref[0, 0:1, off:off + L] = cls
        out_ref[0, 1:2, off:off + L] = jnp.where(keep, tx, 0.0)
        out_ref[0, 2:3, off:off + L] = jnp.where(keep, ty, 0.0)
        out_ref[0, 3:4, off:off + L] = jnp.where(keep, tw, 0.0)
        out_ref[0, 4:5, off:off + L] = jnp.where(keep, th, 0.0)


def _bf16_split3(x):
    def trunc(v):
        bits = jax.lax.bitcast_convert_type(v, jnp.uint32)
        return jax.lax.bitcast_convert_type(
            bits & jnp.uint32(0xFFFF0000), jnp.float32)
    hi = trunc(x)
    r = x - hi
    mid = trunc(r)
    lo = r - mid
    return hi, mid, lo


def kernel(anchors, gt_boxes, img_info, num_gt_boxes):
    N = anchors.shape[0]
    B, G = gt_boxes.shape[0], gt_boxes.shape[1]
    L = 2048
    NP = ((N + L - 1) // L) * L
    GS = ((G + 7) // 8) * 8
    if GS == G:
        GS = G + 8

    pad = jnp.tile(jnp.array([[0.0, 0.0, 2e9, 2e9]], jnp.float32),
                   (NP - N, 1))
    aT = jnp.concatenate([anchors.astype(jnp.float32), pad], axis=0).T
    aT = jnp.concatenate([aT, jnp.zeros((4, NP), jnp.float32)], axis=0)

    num = num_gt_boxes.astype(jnp.int32)
    img = img_info.astype(jnp.float32)

    gtp = jnp.pad(gt_boxes.astype(jnp.float32),
                  ((0, 0), (0, GS - G), (0, 8 - gt_boxes.shape[2])))
    valid = (jnp.arange(GS)[None, :] < num[:, None])[..., None]
    gts = jnp.where(valid, gtp, jnp.float32(-1e8))

    hi, mid, lo = _bf16_split3(gtp[:, :, :4])
    rows = [hi[:, :, 0], mid[:, :, 0], lo[:, :, 0],
            hi[:, :, 1], mid[:, :, 1], lo[:, :, 1],
            hi[:, :, 2], mid[:, :, 2], lo[:, :, 2],
            hi[:, :, 3], mid[:, :, 3], lo[:, :, 3],
            gtp[:, :, 4]]
    gtd = jnp.stack(rows, axis=1)
    gtd = jnp.pad(gtd, ((0, 0), (0, 16 - gtd.shape[1]), (0, 0)))
    gtd = gtd.astype(jnp.bfloat16)

    out = pl.pallas_call(
        functools.partial(_body, NP=NP, L=L, GS=GS),
        grid=(B,),
        in_specs=[
            pl.BlockSpec((8, NP), lambda b: (0, 0)),
            pl.BlockSpec((1, GS, 8), lambda b: (b, 0, 0)),
            pl.BlockSpec((1, 16, GS), lambda b: (b, 0, 0)),
            pl.BlockSpec(memory_space=pltpu.SMEM),
            pl.BlockSpec(memory_space=pltpu.SMEM),
        ],
        out_specs=pl.BlockSpec((1, 8, NP), lambda b: (b, 0, 0)),
        out_shape=jax.ShapeDtypeStruct((B, 8, NP), jnp.float32),
        compiler_params=pltpu.CompilerParams(
            dimension_semantics=("parallel",)),
    )(aT, gts, gtd, num, img)

    cls = out[:, 0, :N]
    reg = jnp.transpose(out[:, 1:5, :N], (0, 2, 1))
    return (cls, reg)


# speedup vs baseline: 1.9719x; 1.1009x over previous
import functools

import jax
import jax.numpy as jnp
from jax.experimental import pallas as pl
from jax.experimental.pallas import tpu as pltpu

FG_IOU = 0.7
BG_IOU = 0.3


def _body(aT_ref, gt_ref, gtd_ref, num_ref, img_ref, out_ref, *, NP, L, GS):
    b = pl.program_id(0)
    img_h = img_ref[0, 0]
    img_w = img_ref[0, 1]
    n_gt = num_ref[b]

    g = gt_ref[0]
    gx1 = g[:, 0:1]
    gy1 = g[:, 1:2]
    gx2 = g[:, 2:3]
    gy2 = g[:, 3:4]
    gw = gx2 - gx1 + 1.0
    gh = gy2 - gy1 + 1.0
    garea = gw * gh
    gd = gtd_ref[0]
    gidx = jax.lax.broadcasted_iota(jnp.int32, (GS, 1), 0)
    gvalid = gidx < n_gt

    lane_i = jax.lax.broadcasted_iota(jnp.int32, (GS, L), 1)
    g_i = jax.lax.broadcasted_iota(jnp.int32, (GS, L), 0)

    nch = NP // L
    acc_cmax = jnp.full((GS, 1), -3.0, jnp.float32)
    acc_carg = jnp.zeros((GS, 1), jnp.int32)
    row_max = []
    row_arg = []

    def anchor_chunk(off):
        ax1 = aT_ref[0:1, off:off + L]
        ay1 = aT_ref[1:2, off:off + L]
        ax2 = aT_ref[2:3, off:off + L]
        ay2 = aT_ref[3:4, off:off + L]
        aw = ax2 - ax1 + 1.0
        ah = ay2 - ay1 + 1.0
        keep = (ax1 >= 0.0) & (ay1 >= 0.0) & (ax2 < img_w) & (ay2 < img_h)
        return ax1, ay1, ax2, ay2, aw, ah, keep

    for c in range(nch):
        off = c * L
        ax1, ay1, ax2, ay2, aw, ah, keep = anchor_chunk(off)
        aarea = aw * ah
        ix1 = jnp.maximum(ax1, gx1)
        iy1 = jnp.maximum(ay1, gy1)
        ix2 = jnp.minimum(ax2, gx2)
        iy2 = jnp.minimum(ay2, gy2)
        iw = jnp.clip(ix2 - ix1 + 1.0, 0.0)
        ih = jnp.clip(iy2 - iy1 + 1.0, 0.0)
        inter = iw * ih
        iou = inter / (aarea + garea - inter)
        ov = jnp.where(keep, iou, -1.0)
        cm = jnp.max(ov, axis=1, keepdims=True)
        carg = jnp.min(jnp.where(ov == cm, lane_i, NP), axis=1,
                       keepdims=True) + off
        better = cm > acc_cmax
        acc_carg = jnp.where(better, carg, acc_carg)
        acc_cmax = jnp.maximum(acc_cmax, cm)
        am = jnp.max(ov, axis=0, keepdims=True)
        aarg = jnp.min(jnp.where(ov == am, g_i, GS), axis=0, keepdims=True)
        row_max.append(am)
        row_arg.append(aarg)

    acc_carg_m = jnp.where(gvalid, acc_carg, -1)

    for c in range(nch):
        off = c * L
        am = row_max[c]
        aarg = row_arg[c]
        eq = (acc_carg_m - off) == lane_i
        best_g = jnp.max(jnp.where(eq, g_i, -1), axis=0, keepdims=True)
        override = best_g >= 0
        arg_f = jnp.where(override, best_g, aarg)
        max_f = jnp.where(override, 2.0, am)
        onehot = (g_i == arg_f).astype(jnp.bfloat16)
        gat = jax.lax.dot_general(gd, onehot, (((1,), (0,)), ((), ())),
                                  preferred_element_type=jnp.float32)
        s_gx1 = gat[0:1, :] + gat[1:2, :] + gat[2:3, :]
        s_gy1 = gat[3:4, :] + gat[4:5, :] + gat[5:6, :]
        s_gx2 = gat[6:7, :] + gat[7:8, :] + gat[8:9, :]
        s_gy2 = gat[9:10, :] + gat[10:11, :] + gat[11:12, :]
        label = gat[12:13, :]
        s_gw = s_gx2 - s_gx1 + 1.0
        s_gh = s_gy2 - s_gy1 + 1.0
        s_gcx = s_gx1 + 0.5 * s_gw
        s_gcy = s_gy1 + 0.5 * s_gh
        ax1, ay1, ax2, ay2, aw, ah, keep = anchor_chunk(off)
        acx = ax1 + 0.5 * aw
        acy = ay1 + 0.5 * ah
        tx = ((s_gcx - acx) / aw) / 0.1
        ty = ((s_gcy - acy) / ah) / 0.1
        tw = jnp.log(s_gw / aw) / 0.2
        th = jnp.log(s_gh / ah) / 0.2
        cls = jnp.where(max_f < FG_IOU, 0.0, label)
        cls = jnp.where((max_f < FG_IOU) & (max_f > BG_IOU), -1.0, cls)
        cls = jnp.where(keep, cls, -1.0)
        out_ref[0, 0:1, off:off + L] = cls
        out_ref[0, 1:2, off:off + L] = jnp.where(keep, tx, 0.0)
        out_ref[0, 2:3, off:off + L] = jnp.where(keep, ty, 0.0)
        out_ref[0, 3:4, off:off + L] = jnp.where(keep, tw, 0.0)
        out_ref[0, 4:5, off:off + L] = jnp.where(keep, th, 0.0)


def _bf16_split3(x):
    def trunc(v):
        bits = jax.lax.bitcast_convert_type(v, jnp.uint32)
        return jax.lax.bitcast_convert_type(
            bits & jnp.uint32(0xFFFF0000), jnp.float32)
    hi = trunc(x)
    r = x - hi
    mid = trunc(r)
    lo = r - mid
    return hi, mid, lo


def kernel(anchors, gt_boxes, img_info, num_gt_boxes):
    N = anchors.shape[0]
    B, G = gt_boxes.shape[0], gt_boxes.shape[1]
    L = 2048
    NP = ((N + L - 1) // L) * L
    GS = ((G + 7) // 8) * 8
    if GS == G:
        GS = G + 8

    pad = jnp.tile(jnp.array([[0.0, 0.0, 2e9, 2e9]], jnp.float32),
                   (NP - N, 1))
    aT = jnp.concatenate([anchors.astype(jnp.float32), pad], axis=0).T
    aT = jnp.concatenate([aT, jnp.zeros((4, NP), jnp.float32)], axis=0)

    num = num_gt_boxes.astype(jnp.int32)
    img = img_info.astype(jnp.float32)

    gtp = jnp.pad(gt_boxes.astype(jnp.float32),
                  ((0, 0), (0, GS - G), (0, 8 - gt_boxes.shape[2])))
    valid = (jnp.arange(GS)[None, :] < num[:, None])[..., None]
    gts = jnp.where(valid, gtp, jnp.float32(-1e8))

    hi, mid, lo = _bf16_split3(gtp[:, :, :4])
    rows = [hi[:, :, 0], mid[:, :, 0], lo[:, :, 0],
            hi[:, :, 1], mid[:, :, 1], lo[:, :, 1],
            hi[:, :, 2], mid[:, :, 2], lo[:, :, 2],
            hi[:, :, 3], mid[:, :, 3], lo[:, :, 3],
            gtp[:, :, 4]]
    gtd = jnp.stack(rows, axis=1)
    gtd = jnp.pad(gtd, ((0, 0), (0, 16 - gtd.shape[1]), (0, 0)))
    gtd = gtd.astype(jnp.bfloat16)

    out = pl.pallas_call(
        functools.partial(_body, NP=NP, L=L, GS=GS),
        grid=(B,),
        in_specs=[
            pl.BlockSpec((8, NP), lambda b: (0, 0)),
            pl.BlockSpec((1, GS, 8), lambda b: (b, 0, 0)),
            pl.BlockSpec((1, 16, GS), lambda b: (b, 0, 0)),
            pl.BlockSpec(memory_space=pltpu.SMEM),
            pl.BlockSpec(memory_space=pltpu.SMEM),
        ],
        out_specs=pl.BlockSpec((1, 8, NP), lambda b: (b, 0, 0)),
        out_shape=jax.ShapeDtypeStruct((B, 8, NP), jnp.float32),
        compiler_params=pltpu.CompilerParams(
            dimension_semantics=("parallel",),
            allow_input_fusion=[True, True, True, False, False]),
    )(aT, gts, gtd, num, img)

    cls = out[:, 0, :N]
    reg = jnp.transpose(out[:, 1:5, :N], (0, 2, 1))
    return (cls, reg)
